# baseline (speedup 1.0000x reference)
import jax
import jax.numpy as jnp
from jax import lax
from jax.experimental import pallas as pl
from jax.experimental.pallas import tpu as pltpu
from jax.experimental.pallas import tpu_sc as plsc

NUM_USERS = 25000
NUM_ITEMS = 75000
NUM_LAYERS = 3
D = 32
B = 128
NCORES = 2
NSUB = 16
GC = 12
CB = GC * B
BODY = 24


def _layer_body(nb, half_pad, pt, e_alloc, ego_hbm, src_hbm, dst_hbm,
                val_hbm, zeros_hbm, out_hbm, srcc, dstc, valc, rows, rowsb,
                acc, semg, sems, semi):
    c = lax.axis_index("c")
    s = lax.axis_index("s")
    h = half_pad // NSUB

    pltpu.sync_copy(zeros_hbm, acc.at[pl.ds(s * h, h)])
    plsc.subcore_barrier()

    tile_e0 = s * pt
    core_e0 = c * e_alloc

    def idx_start(ci, bufset):
        e0 = tile_e0 + ci * CB
        pltpu.async_copy(src_hbm.at[pl.ds(e0, CB)], srcc[bufset], semi)
        pltpu.async_copy(dst_hbm.at[pl.ds(core_e0 + e0, CB)], dstc[bufset],
                         semi)
        pltpu.async_copy(val_hbm.at[pl.ds(core_e0 + e0, CB)], valc[bufset],
                         semi)

    def idx_wait(ci, bufset):
        e0 = tile_e0 + ci * CB
        pltpu.make_async_copy(src_hbm.at[pl.ds(e0, CB)], srcc[bufset],
                              semi).wait()
        pltpu.make_async_copy(dst_hbm.at[pl.ds(core_e0 + e0, CB)],
                              dstc[bufset], semi).wait()
        pltpu.make_async_copy(val_hbm.at[pl.ds(core_e0 + e0, CB)],
                              valc[bufset], semi).wait()

    def gather_start(lj, bufset, gslot):
        pltpu.async_copy(ego_hbm.at[srcc[bufset].at[pl.ds(lj * B, B)]],
                         rows[gslot], semg[gslot])

    def gather_wait(lj, bufset, gslot):
        pltpu.make_async_copy(ego_hbm.at[srcc[bufset].at[pl.ds(lj * B, B)]],
                              rows[gslot], semg[gslot]).wait()

    def scatter_start(lj, bufset, sslot):
        pltpu.async_copy(rowsb[sslot],
                         acc.at[dstc[bufset].at[pl.ds(lj * B, B)]],
                         sems[sslot], add=True)

    def scatter_wait(sslot):
        pltpu.make_async_copy(rowsb[sslot],
                              acc.at[dstc[0].at[pl.ds(0, B)]],
                              sems[sslot]).wait()

    def compute_block(lj, bufset, gslot, sslot):
        def body16(k, carry):
            base = lj * B + k * 16
            sc = valc[bufset][pl.ds(base, 16)]
            for u in range(16):
                e = k * 16 + u
                sv = sc[u]
                rowsb[sslot][e, pl.ds(0, 16)] = rows[gslot][e, pl.ds(0, 16)] * sv
                rowsb[sslot][e, pl.ds(16, 16)] = (
                    rows[gslot][e, pl.ds(16, 16)] * sv)
            return carry

        lax.fori_loop(0, B // 16, body16, 0)

    idx_start(0, 0)
    idx_wait(0, 0)
    for sslot in range(3):
        def zero_rows(k, carry, sslot=sslot):
            rowsb[sslot][k, pl.ds(0, 16)] = jnp.zeros((16,), jnp.float32)
            rowsb[sslot][k, pl.ds(16, 16)] = jnp.zeros((16,), jnp.float32)
            return carry

        lax.fori_loop(0, B, zero_rows, 0)
        scatter_start(sslot, 0, sslot)

    gather_start(0, 0, 0)
    gather_start(1, 0, 1)

    def body(q, carry):
        for p in range(BODY):
            bufset = (p // GC) % 2
            lj = p % GC
            gslot = p % 2
            sslot = p % 3
            if p == 3:
                idx_start(2 * q + 1, 1)
            if p == 10:
                idx_wait(2 * q + 1, 1)
            if p == 15:
                idx_start(2 * q + 2, 0)
            if p == 22:
                idx_wait(2 * q + 2, 0)
            gather_wait(lj, bufset, gslot)
            scatter_wait(sslot)
            compute_block(lj, bufset, gslot, sslot)
            scatter_start(lj, bufset, sslot)
            nbufset = ((p + 2) // GC) % 2
            gather_start((p + 2) % GC, nbufset, gslot)
        return carry

    lax.fori_loop(0, nb // BODY, body, 0)

    gather_wait(0, 0, 0)
    gather_wait(1, 0, 1)
    scatter_wait(0)
    scatter_wait(1)
    scatter_wait(2)

    plsc.subcore_barrier()
    pltpu.sync_copy(acc.at[pl.ds(s * h, h)],
                    out_hbm.at[pl.ds(c * half_pad + s * h, h)])


def _make_layer(n_pad, e_pad, e_alloc):
    half_pad = n_pad // NCORES
    pt = e_pad // NSUB
    nb = pt // B
    mesh = plsc.VectorSubcoreMesh(core_axis_name="c", subcore_axis_name="s")

    def body(ego_hbm, src_hbm, dst_hbm, val_hbm, zeros_hbm, out_hbm,
             srcc0, srcc1, dstc0, dstc1, valc0, valc1,
             rows0, rows1, rowsb0, rowsb1, rowsb2, acc,
             semg0, semg1, sems0, sems1, sems2, semi):
        _layer_body(nb, half_pad, pt, e_alloc, ego_hbm, src_hbm, dst_hbm,
                    val_hbm, zeros_hbm, out_hbm, (srcc0, srcc1),
                    (dstc0, dstc1), (valc0, valc1), (rows0, rows1),
                    (rowsb0, rowsb1, rowsb2), acc, (semg0, semg1),
                    (sems0, sems1, sems2), semi)

    return pl.kernel(
        body,
        out_type=jax.ShapeDtypeStruct((n_pad, D), jnp.float32),
        mesh=mesh,
        scratch_types=[
            pltpu.VMEM((CB,), jnp.int32),
            pltpu.VMEM((CB,), jnp.int32),
            pltpu.VMEM((CB,), jnp.int32),
            pltpu.VMEM((CB,), jnp.int32),
            pltpu.VMEM((CB,), jnp.float32),
            pltpu.VMEM((CB,), jnp.float32),
            pltpu.VMEM((B, D), jnp.float32),
            pltpu.VMEM((B, D), jnp.float32),
            pltpu.VMEM((B, D), jnp.float32),
            pltpu.VMEM((B, D), jnp.float32),
            pltpu.VMEM((B, D), jnp.float32),
            pltpu.VMEM_SHARED((n_pad // NCORES, D), jnp.float32),
            pltpu.SemaphoreType.DMA,
            pltpu.SemaphoreType.DMA,
            pltpu.SemaphoreType.DMA,
            pltpu.SemaphoreType.DMA,
            pltpu.SemaphoreType.DMA,
            pltpu.SemaphoreType.DMA,
        ],
        compiler_params=pltpu.CompilerParams(use_tc_tiling_on_sc=False),
        name="lightgcn_spmm_layer",
    )


def _mean_body(e0, e1, e2, e3, out):
    out[...] = (e0[...] + e1[...] + e2[...] + e3[...]) * 0.25


def _mean4(egos, n):
    rows = n * D // 128
    blk = 1000
    grid = rows // blk
    flat = [e.reshape(rows, 128) for e in egos]
    spec = pl.BlockSpec((blk, 128), lambda i: (i, 0))
    out = pl.pallas_call(
        _mean_body,
        out_shape=jax.ShapeDtypeStruct((rows, 128), jnp.float32),
        grid=(grid,),
        in_specs=[spec] * 4,
        out_specs=spec,
    )(*flat)
    return out.reshape(n, D)


def kernel(adj_indices, adj_values, user_emb, item_emb):
    n = user_emb.shape[0] + item_emb.shape[0]
    row_chunk = NCORES * NSUB * 8
    n_pad = ((n + row_chunk - 1) // row_chunk) * row_chunk
    half = n_pad // NCORES
    e = adj_values.shape[0]
    chunk = NSUB * B * BODY
    e_pad = ((e + chunk - 1) // chunk) * chunk
    e_alloc = e_pad + CB * NSUB

    dst = adj_indices[0]
    src = adj_indices[1]
    pad = e_alloc - e
    dst = jnp.pad(dst, (0, pad))
    src = jnp.pad(src, (0, pad))
    val = jnp.pad(adj_values, (0, pad))
    in0 = dst < half
    idx0 = jnp.where(in0, dst, 0)
    idx1 = jnp.where(in0, 0, dst - half)
    val0 = jnp.where(in0, val, 0.0)
    val1 = jnp.where(in0, 0.0, val)
    dst2 = jnp.concatenate([idx0, idx1]).astype(jnp.int32)
    val2 = jnp.concatenate([val0, val1])
    zeros = jnp.zeros((half // NSUB, D), jnp.float32)

    ego0 = jnp.concatenate(
        [user_emb, item_emb,
         jnp.zeros((n_pad - n, D), jnp.float32)], axis=0)
    layer = _make_layer(n_pad, e_pad, e_alloc)
    ego1 = layer(ego0, src, dst2, val2, zeros)
    ego2 = layer(ego1, src, dst2, val2, zeros)
    ego3 = layer(ego2, src, dst2, val2, zeros)

    final = _mean4([x[:n] for x in (ego0, ego1, ego2, ego3)], n)
    nu = user_emb.shape[0]
    return (final[:nu], final[nu:])
